# baseline (speedup 1.0000x reference)
import functools

import jax
import jax.numpy as jnp
from jax import lax
from jax.experimental import pallas as pl
from jax.experimental.pallas import tpu as pltpu
from jax.experimental.pallas import tpu_sc as plsc

_BATCH = 4096
_N_FIELDS = 26
_CARD = 200
_OUT_COLS = _N_FIELDS * _CARD

_info = plsc.get_sparse_core_info()
_NC, _NS, _L = _info.num_cores, _info.num_subcores, _info.num_lanes
_NW = _NC * _NS
_ROWS_PER_W = _BATCH // _NW
_CHUNK_ROWS = 8
_CHUNKS = _ROWS_PER_W // _CHUNK_ROWS
_XW = _CHUNK_ROWS * _N_FIELDS
_XV = _XW // _L


@functools.partial(
    pl.kernel,
    out_type=jax.ShapeDtypeStruct((_BATCH, _OUT_COLS), jnp.int32),
    mesh=plsc.VectorSubcoreMesh(core_axis_name="c", subcore_axis_name="s"),
    compiler_params=pltpu.CompilerParams(needs_layout_passes=False),
    scratch_types=[
        pltpu.VMEM((_ROWS_PER_W * _N_FIELDS,), jnp.int32),
        pltpu.VMEM((_CHUNK_ROWS, _OUT_COLS), jnp.int32),
        pltpu.VMEM((_CHUNK_ROWS, _OUT_COLS), jnp.int32),
        pltpu.VMEM((_CHUNK_ROWS, _OUT_COLS), jnp.int32),
        pltpu.SemaphoreType.DMA,
        pltpu.SemaphoreType.DMA,
        pltpu.SemaphoreType.DMA,
    ],
)
def _onehot_sc(x_hbm, z_hbm, out_hbm, xv, buf0, buf1, buf2, sem0, sem1, sem2):
    wid = lax.axis_index("s") * _NC + lax.axis_index("c")
    row0 = wid * _ROWS_PER_W
    bufs = (buf0, buf1, buf2)
    sems = (sem0, sem1, sem2)

    ones = jnp.ones((_L,), jnp.int32)
    zeros = jnp.zeros((_L,), jnp.int32)
    iota = lax.iota(jnp.int32, _L)

    xl = pltpu.async_copy(
        x_hbm.at[pl.ds(row0 * _N_FIELDS, _ROWS_PER_W * _N_FIELDS)], xv, sem0)
    pltpu.sync_copy(z_hbm, buf1)
    pltpu.sync_copy(z_hbm, buf2)
    xl.wait()
    pltpu.sync_copy(z_hbm, buf0)

    rowv, colv = [], []
    for v in range(_XV):
        p = v * _L + iota
        r = p // _N_FIELDS
        rowv.append(r)
        colv.append((p - r * _N_FIELDS) * _CARD)

    dmas = []
    for c in range(_CHUNKS):
        dmas.append(pltpu.async_copy(
            buf0, out_hbm.at[pl.ds(row0 + c * _CHUNK_ROWS, _CHUNK_ROWS)],
            sems[c % 3]))
    for d in dmas:
        d.wait()


def kernel(x):
    z = jnp.zeros((_CHUNK_ROWS, _OUT_COLS), jnp.int32)
    return _onehot_sc(x.reshape(-1), z)


# speedup vs baseline: 1.0123x; 1.0123x over previous
import functools

import jax
import jax.numpy as jnp
from jax import lax
from jax.experimental import pallas as pl
from jax.experimental.pallas import tpu as pltpu
from jax.experimental.pallas import tpu_sc as plsc

_BATCH = 4096
_N_FIELDS = 26
_CARD = 200
_OUT_COLS = _N_FIELDS * _CARD

_info = plsc.get_sparse_core_info()
_NC, _NS, _L = _info.num_cores, _info.num_subcores, _info.num_lanes
_NW = _NC * _NS
_ROWS_PER_W = _BATCH // _NW
_CHUNK_ROWS = 8
_CHUNKS = _ROWS_PER_W // _CHUNK_ROWS
_XW = _CHUNK_ROWS * _N_FIELDS
_XV = _XW // _L


@functools.partial(
    pl.kernel,
    out_type=jax.ShapeDtypeStruct((_BATCH, _OUT_COLS), jnp.int32),
    mesh=plsc.VectorSubcoreMesh(core_axis_name="c", subcore_axis_name="s"),
    compiler_params=pltpu.CompilerParams(needs_layout_passes=False),
    scratch_types=[
        pltpu.VMEM((_ROWS_PER_W * _N_FIELDS,), jnp.int32),
        pltpu.VMEM_SHARED((_NS, _CHUNK_ROWS, _OUT_COLS), jnp.int32),
        pltpu.SemaphoreType.DMA,
        pltpu.SemaphoreType.DMA,
        pltpu.SemaphoreType.DMA,
    ],
)
def _onehot_sc(x_hbm, z_hbm, out_hbm, xv, shared, sem0, sem1, sem2):
    wid = lax.axis_index("s") * _NC + lax.axis_index("c")
    sid = lax.axis_index("s")
    row0 = wid * _ROWS_PER_W
    sems = (sem0, sem1, sem2)

    ones = jnp.ones((_L,), jnp.int32)
    zeros = jnp.zeros((_L,), jnp.int32)
    iota = lax.iota(jnp.int32, _L)

    xl = pltpu.async_copy(
        x_hbm.at[pl.ds(row0 * _N_FIELDS, _ROWS_PER_W * _N_FIELDS)], xv, sem0)
    pltpu.sync_copy(z_hbm, shared.at[sid])
    xl.wait()

    rowv, colv = [], []
    for v in range(_XV):
        p = v * _L + iota
        r = p // _N_FIELDS
        rowv.append(r)
        colv.append((p - r * _N_FIELDS) * _CARD)

    dmas = []
    for c in range(_CHUNKS):
        dmas.append(pltpu.async_copy(
            shared.at[sid], out_hbm.at[pl.ds(row0 + c * _CHUNK_ROWS, _CHUNK_ROWS)],
            sems[c % 3]))
    for d in dmas:
        d.wait()


def kernel(x):
    z = jnp.zeros((_CHUNK_ROWS, _OUT_COLS), jnp.int32)
    return _onehot_sc(x.reshape(-1), z)


# speedup vs baseline: 1.3393x; 1.3230x over previous
import functools

import jax
import jax.numpy as jnp
from jax import lax
from jax.experimental import pallas as pl
from jax.experimental.pallas import tpu as pltpu

_BATCH = 4096
_N_FIELDS = 26
_CARD = 200
_OUT_COLS = _N_FIELDS * _CARD
_LANES = 128
_NWIN = (_OUT_COLS + _LANES - 1) // _LANES
_R = 512


def _body(x_ref, o_ref):
    y = x_ref[...] + _CARD * lax.broadcasted_iota(jnp.int32, (1, _N_FIELDS), 1)
    for j in range(_NWIN):
        lo = j * _LANES
        width = min(_LANES, _OUT_COLS - lo)
        i0 = lo // _CARD
        i1 = min(_N_FIELDS - 1, (lo + width - 1) // _CARD)
        c = lo + lax.broadcasted_iota(jnp.int32, (_R, width), 1)
        m = y[:, i0:i0 + 1] == c
        if i1 != i0:
            m = m | (y[:, i1:i1 + 1] == c)
        o_ref[:, lo:lo + width] = m.astype(jnp.int32)


@jax.jit
def _onehot_tc(x):
    return pl.pallas_call(
        _body,
        grid=(_BATCH // _R,),
        in_specs=[pl.BlockSpec((_R, _N_FIELDS), lambda i: (i, 0))],
        out_specs=pl.BlockSpec((_R, _OUT_COLS), lambda i: (i, 0)),
        out_shape=jax.ShapeDtypeStruct((_BATCH, _OUT_COLS), jnp.int32),
        compiler_params=pltpu.CompilerParams(
            dimension_semantics=("arbitrary",)),
    )(x)


def kernel(x):
    return _onehot_tc(x)
